# baseline (speedup 1.0000x reference)
import jax
import jax.numpy as jnp
from jax import lax
from jax.experimental import pallas as pl
from jax.experimental.pallas import tpu as pltpu
from jax.experimental.pallas import tpu_sc as plsc

B = 100000
NLEV = 4
D = 32
DOUT = NLEV * D
C = 640
NCHUNK = 5
SPAN = C * NCHUNK
NW = 32


def _sc_body(idx_all, t0, t1, t2, t3, out,
             iv0, iv1, iv2, iv3, r0, r1, r2, r3, isem, gsem, wsem):
    ivs = (iv0, iv1, iv2, iv3)
    rows = (r0, r1, r2, r3)
    tables = (t0, t1, t2, t3)
    wid = lax.axis_index("s") * 2 + lax.axis_index("c")
    base = pl.multiple_of(jnp.minimum(wid * SPAN, B - SPAN), 8)
    ih = [pltpu.async_copy(idx_all.at[pl.ds(lvl * B + base, SPAN)],
                           ivs[lvl], isem)
          for lvl in range(NLEV)]
    for h in ih:
        h.wait()
    wh = [None] * NLEV
    for it in range(NCHUNK):
        gh = []
        for lvl in range(NLEV):
            if wh[lvl] is not None:
                wh[lvl].wait()
            gh.append(pltpu.async_copy(
                tables[lvl].at[ivs[lvl].at[pl.ds(it * C, C)]],
                rows[lvl], gsem))
        for lvl in range(NLEV):
            gh[lvl].wait()
            wh[lvl] = pltpu.async_copy(
                rows[lvl],
                out.at[pl.ds(base + it * C, C), pl.ds(lvl * D, D)],
                wsem)
    for lvl in range(NLEV):
        wh[lvl].wait()


def kernel(code_levels, table_0, table_1, table_2, table_3):
    idx_all = code_levels.T.reshape(-1)
    mesh = plsc.VectorSubcoreMesh(core_axis_name="c", subcore_axis_name="s")
    run = pl.kernel(
        _sc_body,
        out_type=jax.ShapeDtypeStruct((B, DOUT), jnp.float32),
        mesh=mesh,
        compiler_params=pltpu.CompilerParams(use_tc_tiling_on_sc=False),
        scratch_types=(
            [pltpu.VMEM((SPAN,), jnp.int32)] * NLEV
            + [pltpu.VMEM((C, D), jnp.float32)] * NLEV
            + [pltpu.SemaphoreType.DMA] * 3
        ),
    )
    return run(idx_all, table_0, table_1, table_2, table_3)
